# Initial kernel scaffold; baseline (speedup 1.0000x reference)
import jax
import jax.numpy as jnp
from jax.experimental import pallas as pl


def kernel(features, w1, b1, w2, b2, w3, b3, proj_w, proj_b, codebook):
    raise NotImplementedError("write your pallas kernel here")


# speedup vs baseline: 1.9588x; 1.9588x over previous
import functools

import jax
import jax.numpy as jnp
from jax.experimental import pallas as pl
from jax.experimental.pallas import tpu as pltpu

B, T, D = 4, 1024, 1024
H = 512
LAT = 1024
K = 8192
CD = 8
KC = 1024

_f32 = jnp.float32
_bf16 = jnp.bfloat16


def _bdot(a_bf, b_bf):
    return jnp.dot(a_bf, b_bf, preferred_element_type=_f32)


def _enc_body(x_ref, w1_ref, b1_ref, w2_ref, b2_ref, w3_ref, b3_ref,
              pw_ref, pb_ref, cbn_ref, tok_ref):
    x = x_ref[0]

    def conv3tap(inp_bf, w_ref, b_row, width):
        y0 = _bdot(inp_bf, w_ref[0])
        y1 = _bdot(inp_bf, w_ref[1])
        y2 = _bdot(inp_bf, w_ref[2])
        zrow = jnp.zeros((1, width), _f32)
        return (jnp.concatenate([zrow, y0[:-1]], axis=0) + y1
                + jnp.concatenate([y2[1:], zrow], axis=0) + b_row)

    h1 = jax.nn.gelu(conv3tap(x, w1_ref, b1_ref[...], H))
    h2 = jax.nn.gelu(conv3tap(h1.astype(_bf16), w2_ref, b2_ref[...], H) + h1)
    z = _bdot(h2.astype(_bf16), w3_ref[...]) + b3_ref[...]
    zp = _bdot(z.astype(_bf16), pw_ref[...]) + pb_ref[...]
    zn = zp / (jnp.sqrt(jnp.sum(zp * zp, axis=1, keepdims=True)) + 1e-8)
    zn_bf = zn.astype(_bf16)

    best_v = jnp.full((T, 1), -jnp.inf, _f32)
    best_i = jnp.zeros((T, 1), jnp.int32)
    for kc in range(0, K, KC):
        s = _bdot(zn_bf, cbn_ref[:, kc:kc + KC])
        m = jnp.max(s, axis=1, keepdims=True)
        idx = jax.lax.broadcasted_iota(jnp.int32, s.shape, 1) + kc
        cand = jnp.min(jnp.where(s == m, idx, K), axis=1, keepdims=True)
        upd = m > best_v
        best_v = jnp.where(upd, m, best_v)
        best_i = jnp.where(upd, cand, best_i)
    tok_ref[0] = best_i


def kernel(features, w1, b1, w2, b2, w3, b3, proj_w, proj_b, codebook):
    x_bf = features.astype(_bf16)
    w1k = jnp.transpose(w1, (2, 1, 0)).astype(_bf16)
    w2k = jnp.transpose(w2, (2, 1, 0)).astype(_bf16)
    w3t = jnp.transpose(w3[:, :, 0], (1, 0)).astype(_bf16)
    pwt = jnp.transpose(proj_w, (1, 0)).astype(_bf16)
    cbn = codebook / (jnp.linalg.norm(codebook, axis=-1, keepdims=True) + 1e-8)
    cbnt = jnp.transpose(cbn, (1, 0)).astype(_bf16)
    b1r = b1.reshape(1, H)
    b2r = b2.reshape(1, H)
    b3r = b3.reshape(1, LAT)
    pbr = proj_b.reshape(1, CD)
    tok = pl.pallas_call(
        _enc_body,
        grid=(B,),
        in_specs=[
            pl.BlockSpec((1, T, D), lambda b: (b, 0, 0)),
            pl.BlockSpec((3, D, H), lambda b: (0, 0, 0)),
            pl.BlockSpec((1, H), lambda b: (0, 0)),
            pl.BlockSpec((3, H, H), lambda b: (0, 0, 0)),
            pl.BlockSpec((1, H), lambda b: (0, 0)),
            pl.BlockSpec((H, LAT), lambda b: (0, 0)),
            pl.BlockSpec((1, LAT), lambda b: (0, 0)),
            pl.BlockSpec((LAT, CD), lambda b: (0, 0)),
            pl.BlockSpec((1, CD), lambda b: (0, 0)),
            pl.BlockSpec((CD, K), lambda b: (0, 0)),
        ],
        out_specs=pl.BlockSpec((1, T, 1), lambda b: (b, 0, 0)),
        out_shape=jax.ShapeDtypeStruct((B, T, 1), jnp.int32),
    )(x_bf, w1k, b1r, w2k, b2r, w3t, b3r, pwt, pbr, cbnt)
    return tok.reshape(B, T)
